# baseline (speedup 1.0000x reference)
import functools
import math

import jax
import jax.numpy as jnp
from jax import lax
from jax.experimental import pallas as pl
from jax.experimental.pallas import tpu as pltpu

B, S, IN, D, F, E, K, L, H, OUT = 2, 512, 128, 1024, 4096, 8, 2, 2, 16, 3
N = B * S
DH = D // H
NT = N // 128
NS = 2 * N + E * 128
NST = NS // 128

_bf = jnp.bfloat16
_f32 = jnp.float32


def _split_bf(a):
    hi = a.astype(_bf)
    lo = (a - hi.astype(_f32)).astype(_bf)
    return hi, lo


def _dot3(a, b, dims=((1,), (1,))):
    ah, al = _split_bf(a)
    bh, bl = _split_bf(b)
    dn = (dims, ((), ()))
    r = lax.dot_general(ah, bh, dn, preferred_element_type=_f32)
    r += lax.dot_general(al, bh, dn, preferred_element_type=_f32)
    r += lax.dot_general(ah, bl, dn, preferred_element_type=_f32)
    return r


def _dot3s(ah, al, bh, bl, dims=((1,), (1,))):
    dn = (dims, ((), ()))
    r = lax.dot_general(ah, bh, dn, preferred_element_type=_f32)
    r += lax.dot_general(al, bh, dn, preferred_element_type=_f32)
    r += lax.dot_general(ah, bl, dn, preferred_element_type=_f32)
    return r


def _gelu(x):
    return 0.5 * x * (1.0 + lax.erf(x * (1.0 / math.sqrt(2.0))))


def _ln(y, g, b):
    m = jnp.mean(y, axis=-1, keepdims=True)
    v = jnp.mean((y - m) ** 2, axis=-1, keepdims=True)
    return (y - m) * lax.rsqrt(v + 1e-5) * g + b


def _emit_hilo(y, o_ref, oh_ref, ol_ref):
    o_ref[...] = y
    hi, lo = _split_bf(y)
    oh_ref[...] = hi
    ol_ref[...] = lo


def _inproj_body(x_ref, w_ref, b_ref, pe_ref, o_ref, oh_ref, ol_ref):
    y = _dot3(x_ref[...], w_ref[...]) + b_ref[...] + pe_ref[...]
    _emit_hilo(y, o_ref, oh_ref, ol_ref)


def _inproj(x2d, w, bias, pe):
    return pl.pallas_call(
        _inproj_body,
        grid=(NT,),
        in_specs=[
            pl.BlockSpec((128, IN), lambda t: (t, 0)),
            pl.BlockSpec((D, IN), lambda t: (0, 0)),
            pl.BlockSpec((1, D), lambda t: (0, 0)),
            pl.BlockSpec((128, D), lambda t: (t % (S // 128), 0)),
        ],
        out_specs=[
            pl.BlockSpec((128, D), lambda t: (t, 0)),
            pl.BlockSpec((128, D), lambda t: (t, 0)),
            pl.BlockSpec((128, D), lambda t: (t, 0)),
        ],
        out_shape=[
            jax.ShapeDtypeStruct((N, D), _f32),
            jax.ShapeDtypeStruct((N, D), _bf),
            jax.ShapeDtypeStruct((N, D), _bf),
        ],
    )(x2d, w, bias, pe)


def _attn_body(hh_ref, hl_ref, wq_ref, wk_ref, wv_ref, bq_ref, bk_ref, bv_ref,
               o_ref):
    hh = hh_ref[0]
    hl = hl_ref[0]
    wqh, wql = _split_bf(wq_ref[0])
    wkh, wkl = _split_bf(wk_ref[0])
    wvh, wvl = _split_bf(wv_ref[0])
    q = _dot3s(hh, hl, wqh, wql) + bq_ref[0]
    k = _dot3s(hh, hl, wkh, wkl) + bk_ref[0]
    v = _dot3s(hh, hl, wvh, wvl) + bv_ref[0]
    s = _dot3(q, k) * (1.0 / math.sqrt(DH))
    m = jnp.max(s, axis=-1, keepdims=True)
    p = jnp.exp(s - m)
    p = p / jnp.sum(p, axis=-1, keepdims=True)
    o = _dot3(p, v, dims=((1,), (0,)))
    o_ref[0, 0] = o


def _attention(hh, hl, wq, wk, wv, bq, bk, bv):
    out = pl.pallas_call(
        _attn_body,
        grid=(B, H),
        in_specs=[
            pl.BlockSpec((1, S, D), lambda b, h: (b, 0, 0)),
            pl.BlockSpec((1, S, D), lambda b, h: (b, 0, 0)),
            pl.BlockSpec((1, DH, D), lambda b, h: (h, 0, 0)),
            pl.BlockSpec((1, DH, D), lambda b, h: (h, 0, 0)),
            pl.BlockSpec((1, DH, D), lambda b, h: (h, 0, 0)),
            pl.BlockSpec((1, 1, DH), lambda b, h: (h, 0, 0)),
            pl.BlockSpec((1, 1, DH), lambda b, h: (h, 0, 0)),
            pl.BlockSpec((1, 1, DH), lambda b, h: (h, 0, 0)),
        ],
        out_specs=pl.BlockSpec((1, 1, S, DH), lambda b, h: (b, h, 0, 0)),
        out_shape=jax.ShapeDtypeStruct((B, H, S, DH), _f32),
    )(hh, hl, wq, wk, wv, bq, bk, bv)
    return out.transpose(0, 2, 1, 3).reshape(N, D)


def _projln_body(a_ref, w_ref, b_ref, r_ref, g_ref, bb_ref,
                 o_ref, oh_ref, ol_ref):
    y = _dot3(a_ref[...], w_ref[...]) + b_ref[...] + r_ref[...]
    y = _ln(y, g_ref[...], bb_ref[...])
    _emit_hilo(y, o_ref, oh_ref, ol_ref)


def _projln(ao, w, bias, resid, g, b):
    return pl.pallas_call(
        _projln_body,
        grid=(NT,),
        in_specs=[
            pl.BlockSpec((128, D), lambda t: (t, 0)),
            pl.BlockSpec((D, D), lambda t: (0, 0)),
            pl.BlockSpec((1, D), lambda t: (0, 0)),
            pl.BlockSpec((128, D), lambda t: (t, 0)),
            pl.BlockSpec((1, D), lambda t: (0, 0)),
            pl.BlockSpec((1, D), lambda t: (0, 0)),
        ],
        out_specs=[
            pl.BlockSpec((128, D), lambda t: (t, 0)),
            pl.BlockSpec((128, D), lambda t: (t, 0)),
            pl.BlockSpec((128, D), lambda t: (t, 0)),
        ],
        out_shape=[
            jax.ShapeDtypeStruct((N, D), _f32),
            jax.ShapeDtypeStruct((N, D), _bf),
            jax.ShapeDtypeStruct((N, D), _bf),
        ],
    )(ao, w, bias, resid, g, b)


def _gate_body(h_ref, wg_ref, bg_ref, ti_ref, tw_ref, aux_ref):
    gl = _dot3(h_ref[...], wg_ref[...]) + bg_ref[...]
    mx = jnp.max(gl, axis=-1, keepdims=True)
    pe = jnp.exp(gl - mx)
    pe = pe / jnp.sum(pe, axis=-1, keepdims=True)
    usage = jnp.mean(pe, axis=0, keepdims=True)
    aux_ref[...] = E * jnp.sum(usage * usage, axis=-1, keepdims=True)
    iot = lax.broadcasted_iota(jnp.int32, gl.shape, 1)
    m1 = jnp.max(gl, axis=-1, keepdims=True)
    i1 = jnp.min(jnp.where(gl == m1, iot, E), axis=-1, keepdims=True)
    gl2 = jnp.where(iot == i1, -1e30, gl)
    m2 = jnp.max(gl2, axis=-1, keepdims=True)
    i2 = jnp.min(jnp.where(gl2 == m2, iot, E), axis=-1, keepdims=True)
    w1 = 1.0 / (1.0 + jnp.exp(m2 - m1))
    w2 = 1.0 - w1
    ti_ref[...] = jnp.concatenate([i1, i2], axis=1)
    tw_ref[...] = jnp.concatenate([w1, w2], axis=1)


def _gate(h, wg, bg):
    return pl.pallas_call(
        _gate_body,
        grid=(1,),
        in_specs=[
            pl.BlockSpec((N, D), lambda i: (0, 0)),
            pl.BlockSpec((E, D), lambda i: (0, 0)),
            pl.BlockSpec((1, E), lambda i: (0, 0)),
        ],
        out_specs=[
            pl.BlockSpec((N, K), lambda i: (0, 0)),
            pl.BlockSpec((N, K), lambda i: (0, 0)),
            pl.BlockSpec((1, 1), lambda i: (0, 0)),
        ],
        out_shape=[
            jax.ShapeDtypeStruct((N, K), jnp.int32),
            jax.ShapeDtypeStruct((N, K), _f32),
            jax.ShapeDtypeStruct((1, 1), _f32),
        ],
    )(h, wg, bg)


def _route(ti, tw):
    e_flat = jnp.concatenate([ti[:, 0], ti[:, 1]])
    w_flat = jnp.concatenate([tw[:, 0], tw[:, 1]])
    t_flat = jnp.concatenate([jnp.arange(N, dtype=jnp.int32)] * 2)
    oh = (e_flat[:, None] == jnp.arange(E, dtype=jnp.int32)[None, :])
    ohi = oh.astype(jnp.int32)
    ranks = jnp.cumsum(ohi, axis=0) - ohi
    rank = jnp.sum(jnp.where(oh, ranks, 0), axis=1)
    counts = jnp.sum(ohi, axis=0)
    padded = ((counts + 127) // 128) * 128
    pad_off = jnp.concatenate([jnp.zeros((1,), jnp.int32),
                               jnp.cumsum(padded)]).astype(jnp.int32)
    dst = pad_off[e_flat] + rank
    tok_idx = jnp.zeros((NS,), jnp.int32).at[dst].set(t_flat)
    wvec = jnp.zeros((NS,), _f32).at[dst].set(w_flat)
    base = jnp.arange(NST, dtype=jnp.int32) * 128
    te = jnp.minimum(jnp.sum(pad_off[None, 1:] <= base[:, None], axis=1), E - 1)
    return te.astype(jnp.int32), tok_idx, wvec


def _ffn1_body(te_ref, idx_ref, hh_ref, hl_ref, w1h_ref, w1l_ref, b1_ref,
               t1h_ref, t1l_ref):
    idx = idx_ref[0, 0]
    iot = lax.broadcasted_iota(jnp.int32, (128, N), 1)
    oh = jnp.where(iot == idx[:, None], 1.0, 0.0).astype(_bf)
    dn = (((1,), (0,)), ((), ()))
    xg = lax.dot_general(oh, hh_ref[...], dn, preferred_element_type=_f32)
    xg += lax.dot_general(oh, hl_ref[...], dn, preferred_element_type=_f32)
    xh, xl = _split_bf(xg)
    t1 = _dot3s(xh, xl, w1h_ref[0], w1l_ref[0]) + b1_ref[0]
    t1 = _gelu(t1)
    hi, lo = _split_bf(t1)
    t1h_ref[...] = hi
    t1l_ref[...] = lo


def _moe_ffn1(te, tok_idx, hh, hl, w1h, w1l, b1s):
    grid_spec = pltpu.PrefetchScalarGridSpec(
        num_scalar_prefetch=1,
        grid=(NST,),
        in_specs=[
            pl.BlockSpec((1, 1, 128), lambda t, te_ref: (t, 0, 0)),
            pl.BlockSpec((N, D), lambda t, te_ref: (0, 0)),
            pl.BlockSpec((N, D), lambda t, te_ref: (0, 0)),
            pl.BlockSpec((1, F, D), lambda t, te_ref: (te_ref[t], 0, 0)),
            pl.BlockSpec((1, F, D), lambda t, te_ref: (te_ref[t], 0, 0)),
            pl.BlockSpec((1, 1, F), lambda t, te_ref: (te_ref[t], 0, 0)),
        ],
        out_specs=[
            pl.BlockSpec((128, F), lambda t, te_ref: (t, 0)),
            pl.BlockSpec((128, F), lambda t, te_ref: (t, 0)),
        ],
    )
    return pl.pallas_call(
        _ffn1_body,
        grid_spec=grid_spec,
        out_shape=[
            jax.ShapeDtypeStruct((NS, F), _bf),
            jax.ShapeDtypeStruct((NS, F), _bf),
        ],
    )(te, tok_idx, hh, hl, w1h, w1l, b1s)


def _ffn2_body(te_ref, t1h_ref, t1l_ref, wv_ref, w2h_ref, w2l_ref, b2_ref,
               yh_ref, yl_ref):
    y = _dot3s(t1h_ref[...], t1l_ref[...], w2h_ref[0], w2l_ref[0]) + b2_ref[0]
    y = y * wv_ref[0, 0][:, None]
    hi, lo = _split_bf(y)
    yh_ref[...] = hi
    yl_ref[...] = lo


def _moe_ffn2(te, t1h, t1l, wvec, w2h, w2l, b2s):
    grid_spec = pltpu.PrefetchScalarGridSpec(
        num_scalar_prefetch=1,
        grid=(NST,),
        in_specs=[
            pl.BlockSpec((128, F), lambda t, te_ref: (t, 0)),
            pl.BlockSpec((128, F), lambda t, te_ref: (t, 0)),
            pl.BlockSpec((1, 1, 128), lambda t, te_ref: (t, 0, 0)),
            pl.BlockSpec((1, D, F), lambda t, te_ref: (te_ref[t], 0, 0)),
            pl.BlockSpec((1, D, F), lambda t, te_ref: (te_ref[t], 0, 0)),
            pl.BlockSpec((1, 1, D), lambda t, te_ref: (te_ref[t], 0, 0)),
        ],
        out_specs=[
            pl.BlockSpec((128, D), lambda t, te_ref: (t, 0)),
            pl.BlockSpec((128, D), lambda t, te_ref: (t, 0)),
        ],
    )
    return pl.pallas_call(
        _ffn2_body,
        grid_spec=grid_spec,
        out_shape=[
            jax.ShapeDtypeStruct((NS, D), _bf),
            jax.ShapeDtypeStruct((NS, D), _bf),
        ],
    )(te, t1h, t1l, wvec, w2h, w2l, b2s)


def _combine_body(yh_ref, yl_ref, idx_ref, r_ref, g_ref, bb_ref,
                  o_ref, oh_ref, ol_ref):
    t = pl.program_id(0)
    idx = idx_ref[...]
    iot = lax.broadcasted_iota(jnp.int32, (128, NS), 0) + t * 128
    oh = jnp.where(iot == idx, 1.0, 0.0).astype(_bf)
    dn = (((1,), (0,)), ((), ()))
    mo = lax.dot_general(oh, yh_ref[...], dn, preferred_element_type=_f32)
    mo += lax.dot_general(oh, yl_ref[...], dn, preferred_element_type=_f32)
    y = _ln(mo + r_ref[...], g_ref[...], bb_ref[...])
    _emit_hilo(y, o_ref, oh_ref, ol_ref)


def _moe_combine(yh, yl, idx_row, resid, g, b):
    return pl.pallas_call(
        _combine_body,
        grid=(NT,),
        in_specs=[
            pl.BlockSpec((NS, D), lambda t: (0, 0)),
            pl.BlockSpec((NS, D), lambda t: (0, 0)),
            pl.BlockSpec((1, NS), lambda t: (0, 0)),
            pl.BlockSpec((128, D), lambda t: (t, 0)),
            pl.BlockSpec((1, D), lambda t: (0, 0)),
            pl.BlockSpec((1, D), lambda t: (0, 0)),
        ],
        out_specs=[
            pl.BlockSpec((128, D), lambda t: (t, 0)),
            pl.BlockSpec((128, D), lambda t: (t, 0)),
            pl.BlockSpec((128, D), lambda t: (t, 0)),
        ],
        out_shape=[
            jax.ShapeDtypeStruct((N, D), _f32),
            jax.ShapeDtypeStruct((N, D), _bf),
            jax.ShapeDtypeStruct((N, D), _bf),
        ],
    )(yh, yl, idx_row, resid, g, b)


def _pool_body(h_ref, hl_ref, wq_ref, wk_ref, wv_ref, bq_ref, bk_ref, bv_ref,
               wo_ref, bo_ref,
               aw1_ref, ab1_ref, ag1_ref, agb1_ref, aw2_ref, ab2_ref,
               ag2_ref, agb2_ref, aw3_ref, ab3_ref,
               pw1_ref, pb1_ref, pg1_ref, pgb1_ref, pw2_ref, pb2_ref,
               pg2_ref, pgb2_ref, pw3_ref, pb3_ref,
               act_ref, prof_ref):
    hl8 = jnp.concatenate(
        [hl_ref[...], jnp.zeros((8 - B, D), _f32)], axis=0)
    q = _dot3(hl8, wq_ref[...]) + bq_ref[...]
    rows = lax.broadcasted_iota(jnp.int32, (H, D), 0)
    cols = lax.broadcasted_iota(jnp.int32, (H, D), 1)
    mask = jnp.where(cols // DH == rows, 1.0, 0.0)
    o_rows = []
    for b in range(B):
        hb = h_ref[b]
        kb = _dot3(hb, wk_ref[...]) + bk_ref[...]
        vb = _dot3(hb, wv_ref[...]) + bv_ref[...]
        qp = mask * q[b:b + 1]
        sc = _dot3(qp, kb) * (1.0 / math.sqrt(DH))
        m = jnp.max(sc, axis=-1, keepdims=True)
        p = jnp.exp(sc - m)
        p = p / jnp.sum(p, axis=-1, keepdims=True)
        o_all = _dot3(p, vb, dims=((1,), (0,)))
        o_rows.append(jnp.sum(o_all * mask, axis=0, keepdims=True))
    o_rows.append(jnp.zeros((8 - B, D), _f32))
    o = jnp.concatenate(o_rows, axis=0)
    pooled = _dot3(o, wo_ref[...]) + bo_ref[...]
    a1 = _ln(_gelu(_dot3(pooled, aw1_ref[...]) + ab1_ref[...]), ag1_ref[...], agb1_ref[...])
    a2 = _ln(_gelu(_dot3(a1, aw2_ref[...]) + ab2_ref[...]), ag2_ref[...], agb2_ref[...])
    act_ref[...] = (_dot3(a2, aw3_ref[...]) + ab3_ref[...])[:B, :OUT]
    p1 = _gelu(_ln(_dot3(pooled, pw1_ref[...]) + pb1_ref[...], pg1_ref[...], pgb1_ref[...]))
    p2 = _gelu(_ln(_dot3(p1, pw2_ref[...]) + pb2_ref[...], pg2_ref[...], pgb2_ref[...]))
    prof_ref[...] = (_dot3(p2, pw3_ref[...]) + pb3_ref[...])[:B, :1]


def _pool_heads(h3, hlast, pool_w, ap, pp):
    ins = [h3, hlast] + pool_w + ap + pp
    specs = [pl.BlockSpec(a.shape, functools.partial(lambda r, i: (0,) * r, a.ndim))
             for a in ins]
    return pl.pallas_call(
        _pool_body,
        grid=(1,),
        in_specs=specs,
        out_specs=[
            pl.BlockSpec((B, OUT), lambda i: (0, 0)),
            pl.BlockSpec((B, 1), lambda i: (0, 0)),
        ],
        out_shape=[
            jax.ShapeDtypeStruct((B, OUT), _f32),
            jax.ShapeDtypeStruct((B, 1), _f32),
        ],
    )(*ins)


def _pad8(a, axis=0):
    pads = [(0, 0)] * a.ndim
    pads[axis] = (0, 8 - a.shape[axis])
    return jnp.pad(a, pads)


def kernel(x, params, pos_enc):
    x2d = x.reshape(N, IN)
    pe = pos_enc[0, :S, :]

    ip = params['in_proj']
    h, hh, hl = _inproj(x2d, ip['w'], ip['b'][None], pe)

    aux_total = jnp.zeros((), _f32)
    for lp in params['layers']:
        at = lp['attn']
        iw = at['in_w']
        wq = iw[:D].reshape(H, DH, D)
        wk = iw[D:2 * D].reshape(H, DH, D)
        wv = iw[2 * D:].reshape(H, DH, D)
        ib = at['in_b']
        bq = ib[:D].reshape(H, 1, DH)
        bk = ib[D:2 * D].reshape(H, 1, DH)
        bv = ib[2 * D:].reshape(H, 1, DH)
        ao = _attention(hh.reshape(B, S, D), hl.reshape(B, S, D),
                        wq, wk, wv, bq, bk, bv)
        h, hh, hl = _projln(ao, at['out']['w'], at['out']['b'][None],
                            h, lp['n1']['g'][None], lp['n1']['b'][None])

        ti, tw, aux = _gate(h, lp['gate']['w'], lp['gate']['b'][None])
        aux_total = aux_total + aux[0, 0]

        w1s = jnp.stack([e['l1']['w'] for e in lp['experts']])
        b1s = jnp.stack([e['l1']['b'] for e in lp['experts']])[:, None, :]
        w2s = jnp.stack([e['l2']['w'] for e in lp['experts']])
        b2s = jnp.stack([e['l2']['b'] for e in lp['experts']])[:, None, :]
        w1h, w1l = _split_bf(w1s)
        w2h, w2l = _split_bf(w2s)
        te, tok_idx, wvec = _route(ti, tw)
        t1h, t1l = _moe_ffn1(te, tok_idx.reshape(NST, 1, 128), hh, hl,
                             w1h, w1l, b1s)
        yh, yl = _moe_ffn2(te, t1h, t1l, wvec.reshape(NST, 1, 128),
                           w2h, w2l, b2s)
        h, hh, hl = _moe_combine(yh, yl, tok_idx.reshape(1, NS), h,
                                 lp['n2']['g'][None], lp['n2']['b'][None])

    pw = params['pool']
    piw = pw['in_w']
    pool_w = [piw[:D], piw[D:2 * D], piw[2 * D:],
              pw['in_b'][None, :D], pw['in_b'][None, D:2 * D], pw['in_b'][None, 2 * D:],
              pw['out']['w'], pw['out']['b'][None]]
    apm = params['action']
    ap = [apm['l1']['w'], apm['l1']['b'][None],
          apm['n1']['g'][None], apm['n1']['b'][None],
          apm['l2']['w'], apm['l2']['b'][None],
          apm['n2']['g'][None], apm['n2']['b'][None],
          _pad8(apm['l3']['w']), _pad8(apm['l3']['b'][None], axis=1)]
    ppm = params['profit']
    pp = [ppm['l1']['w'], ppm['l1']['b'][None],
          ppm['n1']['g'][None], ppm['n1']['b'][None],
          ppm['l2']['w'], ppm['l2']['b'][None],
          ppm['n2']['g'][None], ppm['n2']['b'][None],
          _pad8(ppm['l3']['w']), _pad8(ppm['l3']['b'][None], axis=1)]

    h3 = h.reshape(B, S, D)
    hlast = h3[:, S - 1, :]
    action, profit = _pool_heads(h3, hlast, pool_w, ap, pp)
    return action, profit, aux_total


# speedup vs baseline: 1.5578x; 1.5578x over previous
import functools
import math

import jax
import jax.numpy as jnp
from jax import lax
from jax.experimental import pallas as pl
from jax.experimental.pallas import tpu as pltpu

B, S, IN, D, F, E, K, L, H, OUT = 2, 512, 128, 1024, 4096, 8, 2, 2, 16, 3
N = B * S
DH = D // H
NT = N // 128
NS = 2 * N + E * 128
NST = NS // 128
FH = F // 2
DHF = D // 2

_bf = jnp.bfloat16
_f32 = jnp.float32
_CT = (((1,), (0,)), ((), ()))
_CTT = (((1,), (1,)), ((), ()))


def _split_bf(a):
    hi = a.astype(_bf)
    lo = (a - hi.astype(_f32)).astype(_bf)
    return hi, lo


def _mm(a, b, dn=_CT):
    return lax.dot_general(a, b, dn, preferred_element_type=_f32)


def _dot1(a, b, dn=_CT):
    return _mm(a.astype(_bf), b.astype(_bf), dn)


def _gelu(x):
    return 0.5 * x * (1.0 + lax.erf(x * (1.0 / math.sqrt(2.0))))


def _ln(y, g, b):
    m = jnp.mean(y, axis=-1, keepdims=True)
    v = jnp.mean((y - m) ** 2, axis=-1, keepdims=True)
    return (y - m) * lax.rsqrt(v + 1e-5) * g + b


def _inproj_body(x_ref, w_ref, b_ref, pe_ref, o_ref, oh_ref):
    y = _dot1(x_ref[...], w_ref[...]) + b_ref[...] + pe_ref[...]
    o_ref[...] = y
    oh_ref[...] = y.astype(_bf)


def _inproj(x2d, wT, bias, pe):
    return pl.pallas_call(
        _inproj_body,
        grid=(NT,),
        in_specs=[
            pl.BlockSpec((128, IN), lambda t: (t, 0)),
            pl.BlockSpec((IN, D), lambda t: (0, 0)),
            pl.BlockSpec((1, D), lambda t: (0, 0)),
            pl.BlockSpec((128, D), lambda t: (t % (S // 128), 0)),
        ],
        out_specs=[
            pl.BlockSpec((128, D), lambda t: (t, 0)),
            pl.BlockSpec((128, D), lambda t: (t, 0)),
        ],
        out_shape=[
            jax.ShapeDtypeStruct((N, D), _f32),
            jax.ShapeDtypeStruct((N, D), _bf),
        ],
    )(x2d, wT, bias, pe)


def _attn_body(hh_ref, w_ref, b_ref, o_ref):
    hh = hh_ref[0]
    z = _mm(hh, w_ref[...].astype(_bf)) + b_ref[...]
    o_heads = []
    for hd in range(H):
        q = z[:, hd * DH:(hd + 1) * DH]
        k = z[:, D + hd * DH:D + (hd + 1) * DH]
        v = z[:, 2 * D + hd * DH:2 * D + (hd + 1) * DH]
        s = _dot1(q, k, _CTT) * (1.0 / math.sqrt(DH))
        m = jnp.max(s, axis=-1, keepdims=True)
        p = jnp.exp(s - m)
        p = p / jnp.sum(p, axis=-1, keepdims=True)
        o_heads.append(_dot1(p, v))
    o_ref[0] = jnp.concatenate(o_heads, axis=-1)


def _attention(hh, wT, bias):
    return pl.pallas_call(
        _attn_body,
        grid=(B,),
        in_specs=[
            pl.BlockSpec((1, S, D), lambda b: (b, 0, 0)),
            pl.BlockSpec((D, 3 * D), lambda b: (0, 0)),
            pl.BlockSpec((1, 3 * D), lambda b: (0, 0)),
        ],
        out_specs=pl.BlockSpec((1, S, D), lambda b: (b, 0, 0)),
        out_shape=jax.ShapeDtypeStruct((B, S, D), _f32),
    )(hh, wT, bias)


def _projln_body(a_ref, w_ref, b_ref, r_ref, g_ref, bb_ref, o_ref, oh_ref):
    y = _dot1(a_ref[...], w_ref[...]) + b_ref[...] + r_ref[...]
    y = _ln(y, g_ref[...], bb_ref[...])
    o_ref[...] = y
    oh_ref[...] = y.astype(_bf)


def _projln(ao, wT, bias, resid, g, b):
    return pl.pallas_call(
        _projln_body,
        grid=(NT,),
        in_specs=[
            pl.BlockSpec((128, D), lambda t: (t, 0)),
            pl.BlockSpec((D, D), lambda t: (0, 0)),
            pl.BlockSpec((1, D), lambda t: (0, 0)),
            pl.BlockSpec((128, D), lambda t: (t, 0)),
            pl.BlockSpec((1, D), lambda t: (0, 0)),
            pl.BlockSpec((1, D), lambda t: (0, 0)),
        ],
        out_specs=[
            pl.BlockSpec((128, D), lambda t: (t, 0)),
            pl.BlockSpec((128, D), lambda t: (t, 0)),
        ],
        out_shape=[
            jax.ShapeDtypeStruct((N, D), _f32),
            jax.ShapeDtypeStruct((N, D), _bf),
        ],
    )(ao, wT, bias, resid, g, b)


def _gate_body(h_ref, wg_ref, bg_ref, ti_ref, tw_ref, aux_ref):
    gl = _dot1(h_ref[...], wg_ref[...]) + bg_ref[...]
    mx = jnp.max(gl, axis=-1, keepdims=True)
    pe = jnp.exp(gl - mx)
    pe = pe / jnp.sum(pe, axis=-1, keepdims=True)
    usage = jnp.mean(pe, axis=0, keepdims=True)
    aux_ref[...] = E * jnp.sum(usage * usage, axis=-1, keepdims=True)
    iot = lax.broadcasted_iota(jnp.int32, gl.shape, 1)
    m1 = jnp.max(gl, axis=-1, keepdims=True)
    i1 = jnp.min(jnp.where(gl == m1, iot, E), axis=-1, keepdims=True)
    gl2 = jnp.where(iot == i1, -1e30, gl)
    m2 = jnp.max(gl2, axis=-1, keepdims=True)
    i2 = jnp.min(jnp.where(gl2 == m2, iot, E), axis=-1, keepdims=True)
    w1 = 1.0 / (1.0 + jnp.exp(m2 - m1))
    w2 = 1.0 - w1
    ti_ref[...] = jnp.concatenate([i1, i2], axis=1)
    tw_ref[...] = jnp.concatenate([w1, w2], axis=1)


def _gate(h, wgT, bg):
    return pl.pallas_call(
        _gate_body,
        grid=(1,),
        in_specs=[
            pl.BlockSpec((N, D), lambda i: (0, 0)),
            pl.BlockSpec((D, E), lambda i: (0, 0)),
            pl.BlockSpec((1, E), lambda i: (0, 0)),
        ],
        out_specs=[
            pl.BlockSpec((N, K), lambda i: (0, 0)),
            pl.BlockSpec((N, K), lambda i: (0, 0)),
            pl.BlockSpec((1, 1), lambda i: (0, 0)),
        ],
        out_shape=[
            jax.ShapeDtypeStruct((N, K), jnp.int32),
            jax.ShapeDtypeStruct((N, K), _f32),
            jax.ShapeDtypeStruct((1, 1), _f32),
        ],
    )(h, wgT, bg)


def _route(ti, tw):
    e_flat = jnp.concatenate([ti[:, 0], ti[:, 1]])
    w_flat = jnp.concatenate([tw[:, 0], tw[:, 1]])
    t_flat = jnp.concatenate([jnp.arange(N, dtype=jnp.int32)] * 2)
    oh = (e_flat[:, None] == jnp.arange(E, dtype=jnp.int32)[None, :])
    ohi = oh.astype(jnp.int32)
    ranks = jnp.cumsum(ohi, axis=0) - ohi
    rank = jnp.sum(jnp.where(oh, ranks, 0), axis=1)
    counts = jnp.sum(ohi, axis=0)
    padded = ((counts + 127) // 128) * 128
    pad_off = jnp.concatenate([jnp.zeros((1,), jnp.int32),
                               jnp.cumsum(padded)]).astype(jnp.int32)
    dst = pad_off[e_flat] + rank
    tok_idx = jnp.zeros((NS,), jnp.int32).at[dst].set(t_flat)
    wvec = jnp.zeros((NS,), _f32).at[dst].set(w_flat)
    base = jnp.arange(NST, dtype=jnp.int32) * 128
    te = jnp.minimum(jnp.sum(pad_off[None, 1:] <= base[:, None], axis=1), E - 1)
    return te.astype(jnp.int32), tok_idx, wvec


def _gather_body(idx_ref, hh_ref, xs_ref):
    idx = idx_ref[0, 0]
    iot = lax.broadcasted_iota(jnp.int32, (128, N), 1)
    oh = jnp.where(iot == idx[:, None], 1.0, 0.0).astype(_bf)
    xg = _mm(oh, hh_ref[...])
    xs_ref[...] = xg.astype(_bf)


def _gather(tok_idx, hh):
    return pl.pallas_call(
        _gather_body,
        grid=(NST,),
        in_specs=[
            pl.BlockSpec((1, 1, 128), lambda t: (t, 0, 0)),
            pl.BlockSpec((N, D), lambda t: (0, 0)),
        ],
        out_specs=pl.BlockSpec((128, D), lambda t: (t, 0)),
        out_shape=jax.ShapeDtypeStruct((NS, D), _bf),
    )(tok_idx, hh)


def _ffn1_body(te_ref, xs_ref, w1_ref, b1_ref, t1_ref, w_s):
    t = pl.program_id(1)

    @pl.when(jnp.logical_or(t == 0, te_ref[t] != te_ref[jnp.maximum(t - 1, 0)]))
    def _():
        w_s[...] = w1_ref[0].astype(_bf)

    t1 = _mm(xs_ref[...], w_s[...]) + b1_ref[0]
    t1_ref[...] = _gelu(t1).astype(_bf)


def _moe_ffn1(te, xs, w1T, b1s):
    grid_spec = pltpu.PrefetchScalarGridSpec(
        num_scalar_prefetch=1,
        grid=(2, NST),
        in_specs=[
            pl.BlockSpec((128, D), lambda f, t, te_ref: (t, 0)),
            pl.BlockSpec((1, D, FH), lambda f, t, te_ref: (te_ref[t], 0, f)),
            pl.BlockSpec((1, 1, FH), lambda f, t, te_ref: (te_ref[t], 0, f)),
        ],
        out_specs=pl.BlockSpec((128, FH), lambda f, t, te_ref: (t, f)),
        scratch_shapes=[pltpu.VMEM((D, FH), _bf)],
    )
    return pl.pallas_call(
        _ffn1_body,
        grid_spec=grid_spec,
        out_shape=jax.ShapeDtypeStruct((NS, F), _bf),
    )(te, xs, w1T, b1s)


def _ffn2_body(te_ref, t1_ref, wv_ref, w2_ref, b2_ref, yh_ref, yl_ref, w_s):
    t = pl.program_id(1)

    @pl.when(jnp.logical_or(t == 0, te_ref[t] != te_ref[jnp.maximum(t - 1, 0)]))
    def _():
        w_s[...] = w2_ref[0].astype(_bf)

    y = _mm(t1_ref[...], w_s[...]) + b2_ref[0]
    y = y * wv_ref[0, 0][:, None]
    hi, lo = _split_bf(y)
    yh_ref[...] = hi
    yl_ref[...] = lo


def _moe_ffn2(te, t1, wvec, w2T, b2s):
    grid_spec = pltpu.PrefetchScalarGridSpec(
        num_scalar_prefetch=1,
        grid=(2, NST),
        in_specs=[
            pl.BlockSpec((128, F), lambda d, t, te_ref: (t, 0)),
            pl.BlockSpec((1, 1, 128), lambda d, t, te_ref: (t, 0, 0)),
            pl.BlockSpec((1, F, DHF), lambda d, t, te_ref: (te_ref[t], 0, d)),
            pl.BlockSpec((1, 1, DHF), lambda d, t, te_ref: (te_ref[t], 0, d)),
        ],
        out_specs=[
            pl.BlockSpec((128, DHF), lambda d, t, te_ref: (t, d)),
            pl.BlockSpec((128, DHF), lambda d, t, te_ref: (t, d)),
        ],
        scratch_shapes=[pltpu.VMEM((F, DHF), _bf)],
    )
    return pl.pallas_call(
        _ffn2_body,
        grid_spec=grid_spec,
        out_shape=[
            jax.ShapeDtypeStruct((NS, D), _bf),
            jax.ShapeDtypeStruct((NS, D), _bf),
        ],
    )(te, t1, wvec, w2T, b2s)


def _combine_body(yh_ref, yl_ref, idx_ref, r_ref, g_ref, bb_ref,
                  o_ref, oh_ref):
    t = pl.program_id(0)
    idx = idx_ref[...]
    iot = lax.broadcasted_iota(jnp.int32, (128, NS), 0) + t * 128
    oh = jnp.where(iot == idx, 1.0, 0.0).astype(_bf)
    mo = _mm(oh, yh_ref[...]) + _mm(oh, yl_ref[...])
    y = _ln(mo + r_ref[...], g_ref[...], bb_ref[...])
    o_ref[...] = y
    oh_ref[...] = y.astype(_bf)


def _moe_combine(yh, yl, idx_row, resid, g, b):
    return pl.pallas_call(
        _combine_body,
        grid=(NT,),
        in_specs=[
            pl.BlockSpec((NS, D), lambda t: (0, 0)),
            pl.BlockSpec((NS, D), lambda t: (0, 0)),
            pl.BlockSpec((1, NS), lambda t: (0, 0)),
            pl.BlockSpec((128, D), lambda t: (t, 0)),
            pl.BlockSpec((1, D), lambda t: (0, 0)),
            pl.BlockSpec((1, D), lambda t: (0, 0)),
        ],
        out_specs=[
            pl.BlockSpec((128, D), lambda t: (t, 0)),
            pl.BlockSpec((128, D), lambda t: (t, 0)),
        ],
        out_shape=[
            jax.ShapeDtypeStruct((N, D), _f32),
            jax.ShapeDtypeStruct((N, D), _bf),
        ],
    )(yh, yl, idx_row, resid, g, b)


def _pool_body(h_ref, hl_ref, wq_ref, wk_ref, wv_ref, bq_ref, bk_ref, bv_ref,
               wo_ref, bo_ref,
               aw1_ref, ab1_ref, ag1_ref, agb1_ref, aw2_ref, ab2_ref,
               ag2_ref, agb2_ref, aw3_ref, ab3_ref,
               pw1_ref, pb1_ref, pg1_ref, pgb1_ref, pw2_ref, pb2_ref,
               pg2_ref, pgb2_ref, pw3_ref, pb3_ref,
               act_ref, prof_ref):
    hl8 = jnp.concatenate(
        [hl_ref[...], jnp.zeros((8 - B, D), _f32)], axis=0)
    q = _dot1(hl8, wq_ref[...]) + bq_ref[...]
    rows = lax.broadcasted_iota(jnp.int32, (H, D), 0)
    cols = lax.broadcasted_iota(jnp.int32, (H, D), 1)
    mask = jnp.where(cols // DH == rows, 1.0, 0.0)
    o_rows = []
    for b in range(B):
        hb = h_ref[b]
        kb = _dot1(hb, wk_ref[...]) + bk_ref[...]
        vb = _dot1(hb, wv_ref[...]) + bv_ref[...]
        qp = mask * q[b:b + 1]
        sc = _dot1(qp, kb, _CTT) * (1.0 / math.sqrt(DH))
        m = jnp.max(sc, axis=-1, keepdims=True)
        p = jnp.exp(sc - m)
        p = p / jnp.sum(p, axis=-1, keepdims=True)
        o_all = _dot1(p, vb)
        o_rows.append(jnp.sum(o_all * mask, axis=0, keepdims=True))
    o_rows.append(jnp.zeros((8 - B, D), _f32))
    o = jnp.concatenate(o_rows, axis=0)
    pooled = _dot1(o, wo_ref[...]) + bo_ref[...]
    a1 = _ln(_gelu(_dot1(pooled, aw1_ref[...]) + ab1_ref[...]), ag1_ref[...], agb1_ref[...])
    a2 = _ln(_gelu(_dot1(a1, aw2_ref[...]) + ab2_ref[...]), ag2_ref[...], agb2_ref[...])
    act_ref[...] = (_dot1(a2, aw3_ref[...]) + ab3_ref[...])[:B, :OUT]
    p1 = _gelu(_ln(_dot1(pooled, pw1_ref[...]) + pb1_ref[...], pg1_ref[...], pgb1_ref[...]))
    p2 = _gelu(_ln(_dot1(p1, pw2_ref[...]) + pb2_ref[...], pg2_ref[...], pgb2_ref[...]))
    prof_ref[...] = (_dot1(p2, pw3_ref[...]) + pb3_ref[...])[:B, :1]


def _pool_heads(h3, hlast, pool_w, ap, pp):
    ins = [h3, hlast] + pool_w + ap + pp
    specs = [pl.BlockSpec(a.shape, functools.partial(lambda r, i: (0,) * r, a.ndim))
             for a in ins]
    return pl.pallas_call(
        _pool_body,
        grid=(1,),
        in_specs=specs,
        out_specs=[
            pl.BlockSpec((B, OUT), lambda i: (0, 0)),
            pl.BlockSpec((B, 1), lambda i: (0, 0)),
        ],
        out_shape=[
            jax.ShapeDtypeStruct((B, OUT), _f32),
            jax.ShapeDtypeStruct((B, 1), _f32),
        ],
    )(*ins)


def _pad8(a, axis=0):
    pads = [(0, 0)] * a.ndim
    pads[axis] = (0, 8 - a.shape[axis])
    return jnp.pad(a, pads)


def kernel(x, params, pos_enc):
    x2d = x.reshape(N, IN)
    pe = pos_enc[0, :S, :]

    ip = params['in_proj']
    h, hh = _inproj(x2d, ip['w'].T, ip['b'][None], pe)

    aux_total = jnp.zeros((), _f32)
    for lp in params['layers']:
        at = lp['attn']
        ao = _attention(hh.reshape(B, S, D), at['in_w'].T, at['in_b'][None])
        ao = ao.reshape(N, D)
        h, hh = _projln(ao, at['out']['w'].T, at['out']['b'][None],
                        h, lp['n1']['g'][None], lp['n1']['b'][None])

        ti, tw, aux = _gate(h, lp['gate']['w'].T, lp['gate']['b'][None])
        aux_total = aux_total + aux[0, 0]

        w1T = jnp.stack([e['l1']['w'].T for e in lp['experts']])
        b1s = jnp.stack([e['l1']['b'] for e in lp['experts']])[:, None, :]
        w2T = jnp.stack([e['l2']['w'].T for e in lp['experts']])
        b2s = jnp.stack([e['l2']['b'] for e in lp['experts']])[:, None, :]
        te, tok_idx, wvec = _route(ti, tw)
        xs = _gather(tok_idx.reshape(NST, 1, 128), hh)
        t1 = _moe_ffn1(te, xs, w1T, b1s)
        yh, yl = _moe_ffn2(te, t1, wvec.reshape(NST, 1, 128), w2T, b2s)
        h, hh = _moe_combine(yh, yl, tok_idx.reshape(1, NS), h,
                             lp['n2']['g'][None], lp['n2']['b'][None])

    pw = params['pool']
    piw = pw['in_w']
    pool_w = [piw[:D].T, piw[D:2 * D].T, piw[2 * D:].T,
              pw['in_b'][None, :D], pw['in_b'][None, D:2 * D], pw['in_b'][None, 2 * D:],
              pw['out']['w'].T, pw['out']['b'][None]]
    apm = params['action']
    ap = [apm['l1']['w'].T, apm['l1']['b'][None],
          apm['n1']['g'][None], apm['n1']['b'][None],
          apm['l2']['w'].T, apm['l2']['b'][None],
          apm['n2']['g'][None], apm['n2']['b'][None],
          _pad8(apm['l3']['w']).T, _pad8(apm['l3']['b'][None], axis=1)]
    ppm = params['profit']
    pp = [ppm['l1']['w'].T, ppm['l1']['b'][None],
          ppm['n1']['g'][None], ppm['n1']['b'][None],
          ppm['l2']['w'].T, ppm['l2']['b'][None],
          ppm['n2']['g'][None], ppm['n2']['b'][None],
          _pad8(ppm['l3']['w']).T, _pad8(ppm['l3']['b'][None], axis=1)]

    h3 = h.reshape(B, S, D)
    hlast = h3[:, S - 1, :]
    action, profit = _pool_heads(h3, hlast, pool_w, ap, pp)
    return action, profit, aux_total
